# Initial kernel scaffold; baseline (speedup 1.0000x reference)
import jax
import jax.numpy as jnp
from jax.experimental import pallas as pl


def kernel(x, W, b):
    raise NotImplementedError("write your pallas kernel here")


# speedup vs baseline: 1.7420x; 1.7420x over previous
import jax
import jax.numpy as jnp
from jax.experimental import pallas as pl

TH = 0.85
BI = 512


def _qlayer_kern(x_blk_ref, x_ref, w_ref, b_ref, out_ref):
    xb = x_blk_ref[:]
    xa = x_ref[:]
    n = xa.shape[0]
    f = xa.shape[1]
    gram = jnp.dot(xb, xa.T, preferred_element_type=jnp.float32)
    fid = gram * gram
    i = pl.program_id(0)
    rows = i * BI + jax.lax.broadcasted_iota(jnp.int32, (BI, n), 0)
    cols = jax.lax.broadcasted_iota(jnp.int32, (BI, n), 1)
    mask = (fid >= TH) & (rows != cols)
    rxs = jnp.sum(xa, axis=1)
    t = jnp.sum(jnp.where(mask, rxs[None, :], 0.0), axis=1)
    has = jnp.any(mask, axis=1)
    scalar = jnp.where(has, t / f, 0.0)
    wsum = jnp.sum(w_ref[:], axis=1)
    out_ref[:] = scalar[:, None] * wsum[None, :] + b_ref[0, :][None, :]


@jax.jit
def kernel(x, W, b):
    n, f = x.shape
    h = W.shape[0]
    b2 = b.reshape(1, h)
    return pl.pallas_call(
        _qlayer_kern,
        grid=(n // BI,),
        in_specs=[
            pl.BlockSpec((BI, f), lambda i: (i, 0)),
            pl.BlockSpec((n, f), lambda i: (0, 0)),
            pl.BlockSpec((h, f), lambda i: (0, 0)),
            pl.BlockSpec((1, h), lambda i: (0, 0)),
        ],
        out_specs=pl.BlockSpec((BI, h), lambda i: (i, 0)),
        out_shape=jax.ShapeDtypeStruct((n, h), jnp.float32),
    )(x, x, W, b2)


# speedup vs baseline: 1.9812x; 1.1373x over previous
import jax
import jax.numpy as jnp
from jax.experimental import pallas as pl

TH = 0.85
BI = 512


def _qlayer_kern(x_blk_ref, x_ref, w_ref, b_ref, out_ref):
    xb = x_blk_ref[:]
    xa = x_ref[:]
    f = xa.shape[1]
    gram = jnp.dot(xb, xa.T, preferred_element_type=jnp.float32)
    c = gram * gram >= TH
    rxs = jnp.sum(xa, axis=1)
    t_d = jnp.sum(jnp.where(c, rxs[None, :], 0.0), axis=1)
    cnt_d = jnp.sum(c.astype(jnp.float32), axis=1)
    sq = jnp.sum(xb * xb, axis=1)
    diag_c = (sq * sq >= TH).astype(jnp.float32)
    rxs_b = jnp.sum(xb, axis=1)
    t = t_d - diag_c * rxs_b
    cnt = cnt_d - diag_c
    scalar = jnp.where(cnt > 0.5, t / f, 0.0)
    wsum = jnp.sum(w_ref[:], axis=1)
    out_ref[:] = scalar[:, None] * wsum[None, :] + b_ref[0, :][None, :]


@jax.jit
def kernel(x, W, b):
    n, f = x.shape
    h = W.shape[0]
    b2 = b.reshape(1, h)
    return pl.pallas_call(
        _qlayer_kern,
        grid=(n // BI,),
        in_specs=[
            pl.BlockSpec((BI, f), lambda i: (i, 0)),
            pl.BlockSpec((n, f), lambda i: (0, 0)),
            pl.BlockSpec((h, f), lambda i: (0, 0)),
            pl.BlockSpec((1, h), lambda i: (0, 0)),
        ],
        out_specs=pl.BlockSpec((BI, h), lambda i: (i, 0)),
        out_shape=jax.ShapeDtypeStruct((n, h), jnp.float32),
    )(x, x, W, b2)
